# baseline (speedup 1.0000x reference)
import functools

import jax
import jax.numpy as jnp
from jax import lax
from jax.experimental import pallas as pl
from jax.experimental.pallas import tpu as pltpu
from jax.experimental.pallas import tpu_sc as plsc

B, N, D = 16, 4096, 256
ROWS = B * N
NW = 32
RPW = ROWS // NW
VPR = N // 16
VPW = RPW // 16
CHUNK = 128
NCHUNK = RPW // CHUNK
NB = 3


@functools.cache
def _make_sc_kernel():
    mesh = plsc.VectorSubcoreMesh(core_axis_name="c", subcore_axis_name="s")

    @functools.partial(
        pl.kernel,
        mesh=mesh,
        out_type=jax.ShapeDtypeStruct((ROWS, D), jnp.float32),
        scratch_types=[
            pltpu.VMEM((N,), jnp.float32),
            pltpu.VMEM((N,), jnp.int32),
            pltpu.VMEM((RPW,), jnp.int32),
            pltpu.VMEM((CHUNK, D), jnp.float32),
            pltpu.VMEM((CHUNK, D), jnp.float32),
            pltpu.VMEM((CHUNK, D), jnp.float32),
            pltpu.SemaphoreType.DMA,
            pltpu.SemaphoreType.DMA,
            pltpu.SemaphoreType.DMA,
            pltpu.SemaphoreType.DMA,
            pltpu.SemaphoreType.DMA,
            pltpu.SemaphoreType.DMA,
        ],
    )
    def sc_kernel(x_hbm, mask_hbm, out_hbm, mbuf, ffbuf, idxbuf,
                  b0, b1, b2, g0, g1, g2, w0, w1, w2):
        w = lax.axis_index("s") * 2 + lax.axis_index("c")
        b = w >> 1
        h = w & 1
        pltpu.sync_copy(mask_hbm.at[pl.ds(b * N, N)], mbuf)
        lanes = lax.broadcasted_iota(jnp.int32, (16,), 0)
        fifteen = jnp.full((16,), 15, jnp.int32)
        neg1 = jnp.full((16,), -1, jnp.int32)

        def scan_body(i, carry):
            mv = mbuf[pl.ds(i * 16, 16)]
            pos = i * 16 + lanes
            valid = jnp.where(mv > 0.9, neg1, pos)
            cm = valid
            for s in (1, 2, 4, 8):
                sh = cm.at[jnp.maximum(lanes - s, 0)].get(
                    mode="promise_in_bounds")
                cm = jnp.maximum(cm, jnp.where(lanes >= s, sh, neg1))
            cm = jnp.maximum(cm, carry)
            ffbuf[pl.ds(i * 16, 16)] = cm
            return cm.at[fifteen].get(mode="promise_in_bounds")

        row_last = lax.fori_loop(0, VPR, scan_body, neg1)
        fallback = jnp.maximum(row_last, 0)
        rowbase = b * N

        def idx_body(i, carry):
            v = ffbuf[pl.ds((h * VPW + i) * 16, 16)]
            fill = jnp.where(v >= 0, v, fallback)
            idxbuf[pl.ds(i * 16, 16)] = rowbase + fill
            return carry

        lax.fori_loop(0, VPW, idx_body, jnp.int32(0))

        bufs = (b0, b1, b2)
        gsems = (g0, g1, g2)
        wsems = (w0, w1, w2)
        gcp = [None] * NCHUNK
        wcp = [None] * NCHUNK
        for c in range(NB):
            gcp[c] = pltpu.async_copy(
                x_hbm.at[idxbuf.at[pl.ds(c * CHUNK, CHUNK)]],
                bufs[c], gsems[c])
        for c in range(NCHUNK):
            k = c % NB
            gcp[c].wait()
            base = (w * NCHUNK + c) * CHUNK
            wcp[c] = pltpu.async_copy(
                bufs[k], out_hbm.at[pl.ds(base, CHUNK)], wsems[k])
            nxt = c + NB
            if nxt < NCHUNK:
                wcp[c].wait()
                gcp[nxt] = pltpu.async_copy(
                    x_hbm.at[idxbuf.at[pl.ds(nxt * CHUNK, CHUNK)]],
                    bufs[k], gsems[k])
        for c in range(NCHUNK - NB, NCHUNK):
            wcp[c].wait()

    return sc_kernel


def kernel(input, mask):
    x2d = input.reshape(ROWS, D)
    out = _make_sc_kernel()(x2d, mask.reshape(ROWS))
    return out.reshape(B, N, D)
